# baseline (speedup 1.0000x reference)
import functools

import jax
import jax.numpy as jnp
from jax import lax
from jax.experimental import pallas as pl
from jax.experimental.pallas import tpu as pltpu
from jax.experimental.pallas import tpu_sc as plsc

B = 4
BSC = 2
BTC = B - BSC
N = 8192
F = 1024

NC = 2
NS = 16
NW = NC * NS
ROWS_PER_W = N // NW
C = 32
NK = ROWS_PER_W // C

_MESH = plsc.VectorSubcoreMesh(core_axis_name="c", subcore_axis_name="s")


@functools.partial(
    pl.kernel,
    mesh=_MESH,
    out_type=jax.ShapeDtypeStruct((BSC, N, F), jnp.float32),
    scratch_types=[
        pltpu.VMEM((2, C, F), jnp.float32),
        pltpu.SemaphoreType.DMA,
        pltpu.SemaphoreType.DMA,
    ],
)
def _sc_broadcast(table_hbm, out_hbm, buf, sem_in, sem_out):
    wid = lax.axis_index("s") * NC + lax.axis_index("c")
    base = wid * ROWS_PER_W

    pltpu.async_copy(table_hbm.at[pl.ds(base, C), :], buf.at[0], sem_in)
    for k in range(NK):
        slot = k % 2
        r0 = base + k * C
        pltpu.make_async_copy(
            table_hbm.at[pl.ds(r0, C), :], buf.at[slot], sem_in
        ).wait()
        if k > 0:
            for b in range(BSC):
                pltpu.make_async_copy(
                    buf.at[1 - slot],
                    out_hbm.at[b, pl.ds(r0 - C, C), :],
                    sem_out,
                ).wait()
        if k + 1 < NK:
            pltpu.async_copy(
                table_hbm.at[pl.ds(r0 + C, C), :], buf.at[1 - slot], sem_in
            )
        for b in range(BSC):
            pltpu.async_copy(
                buf.at[slot], out_hbm.at[b, pl.ds(r0, C), :], sem_out
            )
    last = base + (NK - 1) * C
    for b in range(BSC):
        pltpu.make_async_copy(
            buf.at[(NK - 1) % 2], out_hbm.at[b, pl.ds(last, C), :], sem_out
        ).wait()


BLK = 1024


def _tc_body(t_ref, o_ref):
    o_ref[...] = jnp.broadcast_to(t_ref[...][None], (BTC, BLK, F))


def _tc_broadcast(table):
    return pl.pallas_call(
        _tc_body,
        grid=(N // BLK,),
        in_specs=[pl.BlockSpec((BLK, F), lambda i: (i, 0))],
        out_specs=pl.BlockSpec((BTC, BLK, F), lambda i: (0, i, 0)),
        out_shape=jax.ShapeDtypeStruct((BTC, N, F), jnp.float32),
    )(table)


def kernel(batch_size, table):
    del batch_size
    sc_part = _sc_broadcast(table)
    tc_part = _tc_broadcast(table)
    return jnp.concatenate([tc_part, sc_part], axis=0)


# speedup vs baseline: 1.8531x; 1.8531x over previous
import functools

import jax
import jax.numpy as jnp
from jax import lax
from jax.experimental import pallas as pl
from jax.experimental.pallas import tpu as pltpu
from jax.experimental.pallas import tpu_sc as plsc

B = 4
N = 8192
F = 1024
NC = 2
NS = 16
NW = NC * NS
ROWS_PER_W = N // NW
C = 32
NK = ROWS_PER_W // C

_MESH = plsc.VectorSubcoreMesh(core_axis_name="c", subcore_axis_name="s")


@functools.partial(
    pl.kernel,
    mesh=_MESH,
    out_type=jax.ShapeDtypeStruct((B, N, F), jnp.float32),
    scratch_types=[
        pltpu.VMEM((2, C, F), jnp.float32),
        pltpu.VMEM_SHARED((NS, 2, C, F), jnp.float32),
        pltpu.SemaphoreType.DMA,
        pltpu.SemaphoreType.DMA,
        pltpu.SemaphoreType.DMA,
        pltpu.SemaphoreType.DMA,
    ],
)
def _sc_broadcast(table_hbm, out_hbm, buf_t, buf_s, sem_in_t, sem_in_s,
                  sem_out_t, sem_out_s):
    sid = lax.axis_index("s")
    wid = sid * NC + lax.axis_index("c")
    base = wid * ROWS_PER_W

    pltpu.async_copy(table_hbm.at[pl.ds(base, C), :], buf_t.at[0], sem_in_t)
    pltpu.async_copy(
        table_hbm.at[pl.ds(base, C), :], buf_s.at[sid, 0], sem_in_s
    )
    for k in range(NK):
        slot = k % 2
        r0 = base + k * C
        pltpu.make_async_copy(
            table_hbm.at[pl.ds(r0, C), :], buf_t.at[slot], sem_in_t
        ).wait()
        pltpu.make_async_copy(
            table_hbm.at[pl.ds(r0, C), :], buf_s.at[sid, slot], sem_in_s
        ).wait()
        if k > 0:
            for b in range(2):
                pltpu.make_async_copy(
                    buf_t.at[1 - slot],
                    out_hbm.at[b, pl.ds(r0 - C, C), :],
                    sem_out_t,
                ).wait()
            for b in range(2, B):
                pltpu.make_async_copy(
                    buf_s.at[sid, 1 - slot],
                    out_hbm.at[b, pl.ds(r0 - C, C), :],
                    sem_out_s,
                ).wait()
        if k + 1 < NK:
            pltpu.async_copy(
                table_hbm.at[pl.ds(r0 + C, C), :], buf_t.at[1 - slot],
                sem_in_t,
            )
            pltpu.async_copy(
                table_hbm.at[pl.ds(r0 + C, C), :], buf_s.at[sid, 1 - slot],
                sem_in_s,
            )
        for b in range(2):
            pltpu.async_copy(
                buf_t.at[slot], out_hbm.at[b, pl.ds(r0, C), :], sem_out_t
            )
        for b in range(2, B):
            pltpu.async_copy(
                buf_s.at[sid, slot], out_hbm.at[b, pl.ds(r0, C), :],
                sem_out_s,
            )
    last = base + (NK - 1) * C
    fslot = (NK - 1) % 2
    for b in range(2):
        pltpu.make_async_copy(
            buf_t.at[fslot], out_hbm.at[b, pl.ds(last, C), :], sem_out_t
        ).wait()
    for b in range(2, B):
        pltpu.make_async_copy(
            buf_s.at[sid, fslot], out_hbm.at[b, pl.ds(last, C), :], sem_out_s
        ).wait()


def kernel(batch_size, table):
    del batch_size
    return _sc_broadcast(table)


# speedup vs baseline: 2.2042x; 1.1894x over previous
import functools

import jax
import jax.numpy as jnp
from jax import lax
from jax.experimental import pallas as pl
from jax.experimental.pallas import tpu as pltpu
from jax.experimental.pallas import tpu_sc as plsc

B = 4
N = 8192
F = 1024
NC = 2
NS = 16
NW = NC * NS
ROWS_PER_W = N // NW
C = 32
NK = ROWS_PER_W // C

_MESH = plsc.VectorSubcoreMesh(core_axis_name="c", subcore_axis_name="s")


@functools.partial(
    pl.kernel,
    mesh=_MESH,
    out_type=jax.ShapeDtypeStruct((B, N, F), jnp.float32),
    scratch_types=[
        pltpu.VMEM((2, C, F), jnp.float32),
        pltpu.SemaphoreType.DMA,
        pltpu.SemaphoreType.DMA,
    ],
)
def _sc_broadcast(table_hbm, out_hbm, buf, sem_in, sem_out):
    wid = lax.axis_index("s") * NC + lax.axis_index("c")
    base = wid * ROWS_PER_W

    pltpu.async_copy(table_hbm.at[pl.ds(base, C), :], buf.at[0], sem_in)
    for k in range(NK):
        slot = k % 2
        r0 = base + k * C
        pltpu.make_async_copy(
            table_hbm.at[pl.ds(r0, C), :], buf.at[slot], sem_in
        ).wait()
        if k > 0:
            for b in range(B):
                pltpu.make_async_copy(
                    buf.at[1 - slot],
                    out_hbm.at[b, pl.ds(r0 - C, C), :],
                    sem_out,
                ).wait()
        if k + 1 < NK:
            pltpu.async_copy(
                table_hbm.at[pl.ds(r0 + C, C), :], buf.at[1 - slot], sem_in
            )
        for b in range(B):
            pltpu.async_copy(
                buf.at[slot], out_hbm.at[b, pl.ds(r0, C), :], sem_out
            )
    last = base + (NK - 1) * C
    for b in range(B):
        pltpu.make_async_copy(
            buf.at[(NK - 1) % 2], out_hbm.at[b, pl.ds(last, C), :], sem_out
        ).wait()


def kernel(batch_size, table):
    del batch_size
    return _sc_broadcast(table)
